# baseline (speedup 1.0000x reference)
import functools
import jax
import jax.numpy as jnp
from jax import lax
from jax.experimental import pallas as pl

BS, N, R, D, GLOVE, TOPN, H = 8, 100, 51, 512, 300, 1024, 8
NN = N * N
DH = D // H
RPAD = 64


def _nt_dot(a, b):
    return lax.dot_general(a, b, (((1,), (1,)), ((), ())),
                           preferred_element_type=jnp.float32)


def _tc_body(rows_ref, vals_ref, feats_ref, arel_ref,
             g0_ref, g1_ref, g2_ref,
             rel_W_ref, rel_b_ref, dic_W_ref, dic_b_ref,
             ln_a_ref, ln_b_ref, lin_W_ref, lin_b_ref,
             fe_W_ref, fe_b_ref,
             Wq_ref, bq_ref, Wk_ref, bk_ref, Wv_ref, bv_ref, Wo_ref, bo_ref,
             anew_ref, pe_ref):
    f = jnp.float32
    rows = rows_ref[0]
    vals = vals_ref[0]
    feats = feats_ref[0]
    arel = arel_ref[0]

    rel_W = rel_W_ref[...]
    dic_W = dic_W_ref[...]
    emb0 = g0_ref[...] @ rel_W + rel_b_ref[...]
    emb1 = g1_ref[...] @ rel_W + rel_b_ref[...]
    emb2 = g2_ref[...] @ rel_W + rel_b_ref[...]
    rg = (emb0 @ dic_W[0:D] + emb1 @ dic_W[D:2 * D]
          + emb2 @ dic_W[2 * D:3 * D] + dic_b_ref[...])

    vmax = jnp.max(vals, axis=0, keepdims=True)
    e = jnp.exp(vals - vmax)
    w = e / jnp.sum(e, axis=0, keepdims=True)

    pe = (rows @ rg) * w

    q = feats @ Wq_ref[...] + bq_ref[...]
    k = pe @ Wk_ref[...] + bk_ref[...]
    v = pe @ Wv_ref[...] + bv_ref[...]

    scale = f(1.0) / jnp.sqrt(f(DH))
    heads = []
    for h in range(H):
        sl = slice(h * DH, (h + 1) * DH)
        s = _nt_dot(q[:, sl], k[:, sl]) * scale
        smax = jnp.max(s, axis=1, keepdims=True)
        es = jnp.exp(s - smax)
        att = es / jnp.sum(es, axis=1, keepdims=True)
        heads.append(att @ v[:, sl])
    mo = jnp.concatenate(heads, axis=1) @ Wo_ref[...] + bo_ref[...]

    mu = jnp.mean(mo, axis=1, keepdims=True)
    xc = mo - mu
    var = jnp.sum(xc * xc, axis=1, keepdims=True) / f(D - 1)
    y = ln_a_ref[...] * xc / (jnp.sqrt(var) + f(1e-6)) + ln_b_ref[...]

    pr = y @ lin_W_ref[...] + lin_b_ref[...]
    fe_W = fe_W_ref[...]
    npre = pr @ fe_W[0:D] + feats @ fe_W[D:2 * D] + fe_b_ref[...]
    anew_ref[0] = _nt_dot(npre, npre) + arel
    pe_ref[0] = pr


def _const2(shape):
    return pl.BlockSpec(shape, lambda b: (0, 0))


@jax.jit
def _tc_call(rows, vals, att_feats, A_rel, g0, g1, g2,
             rel_W, rel_b, dic_W, dic_b, ln_a, ln_b, lin_W, lin_b,
             fe_W, fe_b, Wq, bq, Wk, bk, Wv, bv, Wo, bo):
    specs = [
        pl.BlockSpec((1, TOPN, RPAD), lambda b: (b, 0, 0)),
        pl.BlockSpec((1, TOPN, 1), lambda b: (b, 0, 0)),
        pl.BlockSpec((1, N, D), lambda b: (b, 0, 0)),
        pl.BlockSpec((1, N, N), lambda b: (b, 0, 0)),
        _const2((RPAD, GLOVE)), _const2((RPAD, GLOVE)), _const2((RPAD, GLOVE)),
        _const2((GLOVE, D)), _const2((1, D)),
        _const2((3 * D, D)), _const2((1, D)),
        _const2((1, D)), _const2((1, D)),
        _const2((D, D)), _const2((1, D)),
        _const2((2 * D, D)), _const2((1, D)),
        _const2((D, D)), _const2((1, D)),
        _const2((D, D)), _const2((1, D)),
        _const2((D, D)), _const2((1, D)),
        _const2((D, D)), _const2((1, D)),
    ]
    out_specs = [
        pl.BlockSpec((1, N, N), lambda b: (b, 0, 0)),
        pl.BlockSpec((1, N, D), lambda b: (b, 0, 0)),
    ]
    out_shapes = [
        jax.ShapeDtypeStruct((BS, N, N), jnp.float32),
        jax.ShapeDtypeStruct((BS, N, D), jnp.float32),
    ]
    return pl.pallas_call(
        _tc_body,
        grid=(BS,),
        in_specs=specs,
        out_specs=out_specs,
        out_shape=out_shapes,
    )(rows, vals, att_feats, A_rel, g0, g1, g2,
      rel_W, rel_b, dic_W, dic_b, ln_a, ln_b, lin_W, lin_b,
      fe_W, fe_b, Wq, bq, Wk, bk, Wv, bv, Wo, bo)


def kernel(A_rel, rel_scores, att_feats, att_mask, vocab_glove, rel_dic,
           rel_W, rel_b, dic_W, dic_b, ln_a, ln_b, lin_W, lin_b,
           fe_W, fe_b, Wq, bq, Wk, bk, Wv, bv, Wo, bo):
    del att_mask
    flat = A_rel.reshape(BS, NN)
    vals, idx = lax.top_k(flat, TOPN)
    rs = rel_scores.reshape(BS, NN, R)
    rows = jnp.take_along_axis(rs, idx[..., None], axis=1)
    rows = jnp.pad(rows, ((0, 0), (0, 0), (0, RPAD - R)))

    g = jnp.take(vocab_glove, rel_dic, axis=0)
    g = jnp.pad(g, ((0, RPAD - R), (0, 0), (0, 0)))
    g0, g1, g2 = g[:, 0], g[:, 1], g[:, 2]

    A_new, pe_rel_reps = _tc_call(
        rows, vals[..., None], att_feats, A_rel, g0, g1, g2,
        rel_W, rel_b.reshape(1, D), dic_W, dic_b.reshape(1, D),
        ln_a.reshape(1, D), ln_b.reshape(1, D), lin_W, lin_b.reshape(1, D),
        fe_W, fe_b.reshape(1, D), Wq, bq.reshape(1, D), Wk, bk.reshape(1, D),
        Wv, bv.reshape(1, D), Wo, bo.reshape(1, D))
    return (A_new, pe_rel_reps)


# speedup vs baseline: 1.0013x; 1.0013x over previous
import functools
import numpy as np
import jax
import jax.numpy as jnp
from jax import lax
from jax.experimental import pallas as pl
from jax.experimental.pallas import tpu as pltpu
from jax.experimental.pallas import tpu_sc as plsc

BS, N, R, D, GLOVE, TOPN, H = 8, 100, 51, 512, 300, 1024, 8
NN = N * N
DH = D // H
RPAD = 64


_IMIN = np.int32(-2147483648)


def _sc_body(a_hbm, rel_hbm, vals_hbm, rows_hbm,
             keys_f, keyA, idxA, keyB, idxB, oval, oidx, idx_g, rows_v, sem):
    wid = lax.axis_index("s") * 2 + lax.axis_index("c")

    @pl.when(wid < BS)
    def _():
        b = wid
        pltpu.sync_copy(a_hbm.at[b], keys_f)
        lane = lax.iota(jnp.int32, 16)
        nv0 = NN // 16

        def init_body(i, cnt):
            x = keys_f[pl.ds(i * 16, 16)]
            bb = lax.bitcast_convert_type(x, jnp.int32)
            k = jnp.where(bb < 0, _IMIN - bb, bb)
            keyA[pl.ds(i * 16, 16)] = k
            idxA[pl.ds(i * 16, 16)] = lane + i * 16
            return cnt + jnp.sum((k >= 0).astype(jnp.int32))

        cnt = lax.fori_loop(0, nv0, init_body, jnp.int32(0))

        t = jnp.full((), _IMIN, jnp.int32)
        in_cnt = jnp.int32(0)
        csize = jnp.int32(NN)
        out_n = jnp.int32(0)
        bufs = [(keyA, idxA), (keyB, idxB)]
        for step, bit in enumerate(range(31, -1, -1)):
            curK, curI = bufs[step % 2]
            nxtK, nxtI = bufs[(step + 1) % 2]
            if bit == 31:
                cand = jnp.zeros((), jnp.int32)
            else:
                cand = t | np.int32(1 << bit)
            dec = (in_cnt + cnt) >= TOPN
            t = jnp.where(dec, cand, t)
            in_cnt = jnp.where(dec, in_cnt, in_cnt + cnt)
            if bit > 0:
                next_cand = t | np.int32(1 << (bit - 1))
            else:
                next_cand = t
            deci = dec.astype(jnp.int32)
            cs_now = csize

            def pass_body(i, carry, curK=curK, curI=curI, nxtK=nxtK,
                          nxtI=nxtI, cand=cand, deci=deci,
                          next_cand=next_cand, cs_now=cs_now):
                kb, ob, cn = carry
                k = curK[pl.ds(i * 16, 16)]
                ix = curI[pl.ds(i * 16, 16)]
                valid = (lane + i * 16) < cs_now
                m = jnp.logical_and(valid, k >= cand)
                keep = jnp.logical_and(valid, m.astype(jnp.int32) == deci)
                ck = plsc.cumsum(keep.astype(jnp.int32))
                pos = kb + ck - 1
                plsc.store_scatter(nxtK, [pos], k, mask=keep)
                plsc.store_scatter(nxtI, [pos], ix, mask=keep)
                app = jnp.logical_and(m, deci == 0)
                ca = plsc.cumsum(app.astype(jnp.int32))
                posa = ob + ca - 1
                plsc.store_scatter(oidx, [posa], ix, mask=app)
                vb = jnp.where(k >= 0, k, _IMIN - k)
                plsc.store_scatter(oval, [posa],
                                   lax.bitcast_convert_type(vb, jnp.float32),
                                   mask=app)
                cn2 = cn + jnp.sum(
                    jnp.logical_and(keep, k >= next_cand).astype(jnp.int32))
                return (kb + jnp.sum(keep.astype(jnp.int32)),
                        ob + jnp.sum(app.astype(jnp.int32)), cn2)

            nv = (cs_now + 15) // 16
            csize, out_n, cnt = lax.fori_loop(
                0, nv, pass_body, (jnp.int32(0), out_n, jnp.int32(0)))

        need = TOPN - in_cnt
        curK, curI = bufs[0]
        cs_fin = csize

        def tie_body(i, carry):
            taken, ob = carry
            ix = curI[pl.ds(i * 16, 16)]
            valid = (lane + i * 16) < cs_fin
            pr = plsc.cumsum(valid.astype(jnp.int32))
            sel = jnp.logical_and(valid, (taken + pr) <= need)
            cs = plsc.cumsum(sel.astype(jnp.int32))
            pos = ob + cs - 1
            plsc.store_scatter(oidx, [pos], ix, mask=sel)
            tvec = jnp.broadcast_to(t, (16,))
            vvec = lax.bitcast_convert_type(
                jnp.where(tvec >= 0, tvec, _IMIN - tvec), jnp.float32)
            plsc.store_scatter(oval, [pos], vvec, mask=sel)
            return (taken + jnp.sum(valid.astype(jnp.int32)),
                    ob + jnp.sum(sel.astype(jnp.int32)))

        lax.fori_loop(0, (cs_fin + 15) // 16, tie_body,
                      (jnp.int32(0), out_n))

        base = b * NN
        for j in range(8):
            for q in range(8):
                v = oidx[pl.ds((j * 8 + q) * 16, 16)] + base
                idx_g[j, pl.ds(q * 16, 16)] = v
        for j in range(8):
            pltpu.async_copy(rel_hbm.at[idx_g.at[j]],
                             rows_v.at[pl.ds(j * 128, 128)], sem).wait()
        pltpu.sync_copy(rows_v, rows_hbm.at[b])
        pltpu.sync_copy(oval, vals_hbm.at[b])


def _sc_call(a2d, rel_pad):
    mesh = plsc.VectorSubcoreMesh(core_axis_name="c", subcore_axis_name="s")
    f = pl.kernel(
        _sc_body,
        out_type=[jax.ShapeDtypeStruct((BS, TOPN), jnp.float32),
                  jax.ShapeDtypeStruct((BS, TOPN, RPAD), jnp.float32)],
        mesh=mesh,
        scratch_types=[
            pltpu.VMEM((NN,), jnp.float32),
            pltpu.VMEM((NN,), jnp.int32), pltpu.VMEM((NN,), jnp.int32),
            pltpu.VMEM((NN,), jnp.int32), pltpu.VMEM((NN,), jnp.int32),
            pltpu.VMEM((TOPN,), jnp.float32), pltpu.VMEM((TOPN,), jnp.int32),
            pltpu.VMEM((8, 128), jnp.int32),
            pltpu.VMEM((TOPN, RPAD), jnp.float32),
            pltpu.SemaphoreType.DMA,
        ])
    return f(a2d, rel_pad)


def _nt_dot(a, b):
    return lax.dot_general(a, b, (((1,), (1,)), ((), ())),
                           preferred_element_type=jnp.float32)


def _tc_body(rows_ref, vals_ref, feats_ref, arel_ref,
             g0_ref, g1_ref, g2_ref,
             rel_W_ref, rel_b_ref, dic_W_ref, dic_b_ref,
             ln_a_ref, ln_b_ref, lin_W_ref, lin_b_ref,
             fe_W_ref, fe_b_ref,
             Wq_ref, bq_ref, Wk_ref, bk_ref, Wv_ref, bv_ref, Wo_ref, bo_ref,
             anew_ref, pe_ref):
    f = jnp.float32
    rows = rows_ref[0]
    vals = vals_ref[0]
    feats = feats_ref[0]
    arel = arel_ref[0]

    rel_W = rel_W_ref[...]
    dic_W = dic_W_ref[...]
    emb0 = g0_ref[...] @ rel_W + rel_b_ref[...]
    emb1 = g1_ref[...] @ rel_W + rel_b_ref[...]
    emb2 = g2_ref[...] @ rel_W + rel_b_ref[...]
    rg = (emb0 @ dic_W[0:D] + emb1 @ dic_W[D:2 * D]
          + emb2 @ dic_W[2 * D:3 * D] + dic_b_ref[...])

    vmax = jnp.max(vals, axis=0, keepdims=True)
    e = jnp.exp(vals - vmax)
    w = e / jnp.sum(e, axis=0, keepdims=True)

    pe = (rows @ rg) * w

    q = feats @ Wq_ref[...] + bq_ref[...]
    k = pe @ Wk_ref[...] + bk_ref[...]
    v = pe @ Wv_ref[...] + bv_ref[...]

    scale = f(1.0) / jnp.sqrt(f(DH))
    heads = []
    for h in range(H):
        sl = slice(h * DH, (h + 1) * DH)
        s = _nt_dot(q[:, sl], k[:, sl]) * scale
        smax = jnp.max(s, axis=1, keepdims=True)
        es = jnp.exp(s - smax)
        att = es / jnp.sum(es, axis=1, keepdims=True)
        heads.append(att @ v[:, sl])
    mo = jnp.concatenate(heads, axis=1) @ Wo_ref[...] + bo_ref[...]

    mu = jnp.mean(mo, axis=1, keepdims=True)
    xc = mo - mu
    var = jnp.sum(xc * xc, axis=1, keepdims=True) / f(D - 1)
    y = ln_a_ref[...] * xc / (jnp.sqrt(var) + f(1e-6)) + ln_b_ref[...]

    pr = y @ lin_W_ref[...] + lin_b_ref[...]
    fe_W = fe_W_ref[...]
    npre = pr @ fe_W[0:D] + feats @ fe_W[D:2 * D] + fe_b_ref[...]
    anew_ref[0] = _nt_dot(npre, npre) + arel
    pe_ref[0] = pr


def _const2(shape):
    return pl.BlockSpec(shape, lambda b: (0, 0))


@jax.jit
def _tc_call(rows, vals, att_feats, A_rel, g0, g1, g2,
             rel_W, rel_b, dic_W, dic_b, ln_a, ln_b, lin_W, lin_b,
             fe_W, fe_b, Wq, bq, Wk, bk, Wv, bv, Wo, bo):
    specs = [
        pl.BlockSpec((1, TOPN, RPAD), lambda b: (b, 0, 0)),
        pl.BlockSpec((1, TOPN, 1), lambda b: (b, 0, 0)),
        pl.BlockSpec((1, N, D), lambda b: (b, 0, 0)),
        pl.BlockSpec((1, N, N), lambda b: (b, 0, 0)),
        _const2((RPAD, GLOVE)), _const2((RPAD, GLOVE)), _const2((RPAD, GLOVE)),
        _const2((GLOVE, D)), _const2((1, D)),
        _const2((3 * D, D)), _const2((1, D)),
        _const2((1, D)), _const2((1, D)),
        _const2((D, D)), _const2((1, D)),
        _const2((2 * D, D)), _const2((1, D)),
        _const2((D, D)), _const2((1, D)),
        _const2((D, D)), _const2((1, D)),
        _const2((D, D)), _const2((1, D)),
        _const2((D, D)), _const2((1, D)),
    ]
    out_specs = [
        pl.BlockSpec((1, N, N), lambda b: (b, 0, 0)),
        pl.BlockSpec((1, N, D), lambda b: (b, 0, 0)),
    ]
    out_shapes = [
        jax.ShapeDtypeStruct((BS, N, N), jnp.float32),
        jax.ShapeDtypeStruct((BS, N, D), jnp.float32),
    ]
    return pl.pallas_call(
        _tc_body,
        grid=(BS,),
        in_specs=specs,
        out_specs=out_specs,
        out_shape=out_shapes,
    )(rows, vals, att_feats, A_rel, g0, g1, g2,
      rel_W, rel_b, dic_W, dic_b, ln_a, ln_b, lin_W, lin_b,
      fe_W, fe_b, Wq, bq, Wk, bk, Wv, bv, Wo, bo)


def kernel(A_rel, rel_scores, att_feats, att_mask, vocab_glove, rel_dic,
           rel_W, rel_b, dic_W, dic_b, ln_a, ln_b, lin_W, lin_b,
           fe_W, fe_b, Wq, bq, Wk, bk, Wv, bv, Wo, bo):
    del att_mask
    flat = A_rel.reshape(BS, NN)
    vals, idx = lax.top_k(flat, TOPN)
    rs = rel_scores.reshape(BS, NN, R)
    rows = jnp.take_along_axis(rs, idx[..., None], axis=1)
    rows = jnp.pad(rows, ((0, 0), (0, 0), (0, RPAD - R)))
    vals = vals

    g = jnp.take(vocab_glove, rel_dic, axis=0)
    g = jnp.pad(g, ((0, RPAD - R), (0, 0), (0, 0)))
    g0, g1, g2 = g[:, 0], g[:, 1], g[:, 2]

    A_new, pe_rel_reps = _tc_call(
        rows, vals[..., None], att_feats, A_rel, g0, g1, g2,
        rel_W, rel_b.reshape(1, D), dic_W, dic_b.reshape(1, D),
        ln_a.reshape(1, D), ln_b.reshape(1, D), lin_W, lin_b.reshape(1, D),
        fe_W, fe_b.reshape(1, D), Wq, bq.reshape(1, D), Wk, bk.reshape(1, D),
        Wv, bv.reshape(1, D), Wo, bo.reshape(1, D))
    return (A_new, pe_rel_reps)


# speedup vs baseline: 1.0755x; 1.0741x over previous
import functools
import numpy as np
import jax
import jax.numpy as jnp
from jax import lax
from jax.experimental import pallas as pl
from jax.experimental.pallas import tpu as pltpu
from jax.experimental.pallas import tpu_sc as plsc

BS, N, R, D, GLOVE, TOPN, H = 8, 100, 51, 512, 300, 1024, 8
NN = N * N
DH = D // H
RPAD = 64


_IMIN = np.int32(-2147483648)


def _sc_body(a_hbm, rel_hbm, vals_hbm, rows_hbm,
             keys_f, keyA, idxA, keyB, idxB, oval, oidx, idx_g, rows_v, sem):
    wid = lax.axis_index("s") * 2 + lax.axis_index("c")

    @pl.when(wid < BS)
    def _():
        b = wid
        pltpu.sync_copy(a_hbm.at[b], keys_f)
        lane = lax.iota(jnp.int32, 16)
        nv0 = NN // 16

        def init_body(i, cnt):
            x = keys_f[pl.ds(i * 16, 16)]
            bb = lax.bitcast_convert_type(x, jnp.int32)
            k = jnp.where(bb < 0, _IMIN - bb, bb)
            keyA[pl.ds(i * 16, 16)] = k
            idxA[pl.ds(i * 16, 16)] = lane + i * 16
            return cnt + jnp.sum((k >= 0).astype(jnp.int32))

        cnt = lax.fori_loop(0, nv0, init_body, jnp.int32(0))

        t = jnp.full((), _IMIN, jnp.int32)
        in_cnt = jnp.int32(0)
        csize = jnp.int32(NN)
        out_n = jnp.int32(0)
        bufs = [(keyA, idxA), (keyB, idxB)]
        for step, bit in enumerate(range(31, -1, -1)):
            curK, curI = bufs[step % 2]
            nxtK, nxtI = bufs[(step + 1) % 2]
            if bit == 31:
                cand = jnp.zeros((), jnp.int32)
            else:
                cand = t | np.int32(1 << bit)
            dec = (in_cnt + cnt) >= TOPN
            t = jnp.where(dec, cand, t)
            in_cnt = jnp.where(dec, in_cnt, in_cnt + cnt)
            if bit > 0:
                next_cand = t | np.int32(1 << (bit - 1))
            else:
                next_cand = t
            deci = dec.astype(jnp.int32)
            cs_now = csize

            def pass_body(i, carry, curK=curK, curI=curI, nxtK=nxtK,
                          nxtI=nxtI, cand=cand, deci=deci,
                          next_cand=next_cand, cs_now=cs_now):
                kb, ob, cn = carry
                k = curK[pl.ds(i * 16, 16)]
                ix = curI[pl.ds(i * 16, 16)]
                valid = (lane + i * 16) < cs_now
                m = jnp.logical_and(valid, k >= cand)
                keep = jnp.logical_and(valid, m.astype(jnp.int32) == deci)
                ck = plsc.cumsum(keep.astype(jnp.int32))
                pos = kb + ck - 1
                plsc.store_scatter(nxtK, [pos], k, mask=keep)
                plsc.store_scatter(nxtI, [pos], ix, mask=keep)
                app = jnp.logical_and(m, deci == 0)
                ca = plsc.cumsum(app.astype(jnp.int32))
                posa = ob + ca - 1
                plsc.store_scatter(oidx, [posa], ix, mask=app)
                vb = jnp.where(k >= 0, k, _IMIN - k)
                plsc.store_scatter(oval, [posa],
                                   lax.bitcast_convert_type(vb, jnp.float32),
                                   mask=app)
                cn2 = cn + jnp.sum(
                    jnp.logical_and(keep, k >= next_cand).astype(jnp.int32))
                return (kb + jnp.sum(keep.astype(jnp.int32)),
                        ob + jnp.sum(app.astype(jnp.int32)), cn2)

            nv = (cs_now + 15) // 16
            csize, out_n, cnt = lax.fori_loop(
                0, nv, pass_body, (jnp.int32(0), out_n, jnp.int32(0)))

        need = TOPN - in_cnt
        curK, curI = bufs[0]
        cs_fin = csize

        def tie_body(i, carry):
            taken, ob = carry
            ix = curI[pl.ds(i * 16, 16)]
            valid = (lane + i * 16) < cs_fin
            pr = plsc.cumsum(valid.astype(jnp.int32))
            sel = jnp.logical_and(valid, (taken + pr) <= need)
            cs = plsc.cumsum(sel.astype(jnp.int32))
            pos = ob + cs - 1
            plsc.store_scatter(oidx, [pos], ix, mask=sel)
            tvec = jnp.broadcast_to(t, (16,))
            vvec = lax.bitcast_convert_type(
                jnp.where(tvec >= 0, tvec, _IMIN - tvec), jnp.float32)
            plsc.store_scatter(oval, [pos], vvec, mask=sel)
            return (taken + jnp.sum(valid.astype(jnp.int32)),
                    ob + jnp.sum(sel.astype(jnp.int32)))

        lax.fori_loop(0, (cs_fin + 15) // 16, tie_body,
                      (jnp.int32(0), out_n))

        base = b * NN
        for j in range(8):
            for q in range(8):
                v = oidx[pl.ds((j * 8 + q) * 16, 16)] + base
                idx_g[j, pl.ds(q * 16, 16)] = v
        for j in range(8):
            pltpu.async_copy(rel_hbm.at[idx_g.at[j]],
                             rows_v.at[pl.ds(j * 128, 128)], sem).wait()
        pltpu.sync_copy(rows_v, rows_hbm.at[b])
        pltpu.sync_copy(oval, vals_hbm.at[b])


def _sc_call(a2d, rel_pad):
    mesh = plsc.VectorSubcoreMesh(core_axis_name="c", subcore_axis_name="s")
    f = pl.kernel(
        _sc_body,
        out_type=[jax.ShapeDtypeStruct((BS, TOPN), jnp.float32),
                  jax.ShapeDtypeStruct((BS, TOPN, RPAD), jnp.float32)],
        mesh=mesh,
        scratch_types=[
            pltpu.VMEM((NN,), jnp.float32),
            pltpu.VMEM((NN,), jnp.int32), pltpu.VMEM((NN,), jnp.int32),
            pltpu.VMEM((NN,), jnp.int32), pltpu.VMEM((NN,), jnp.int32),
            pltpu.VMEM((TOPN,), jnp.float32), pltpu.VMEM((TOPN,), jnp.int32),
            pltpu.VMEM((8, 128), jnp.int32),
            pltpu.VMEM((TOPN, RPAD), jnp.float32),
            pltpu.SemaphoreType.DMA,
        ])
    return f(a2d, rel_pad)


def _nt_dot(a, b):
    return lax.dot_general(a, b, (((1,), (1,)), ((), ())),
                           preferred_element_type=jnp.float32)


def _tc_body(rows_ref, vals_ref, feats_ref, arel_ref,
             g0_ref, g1_ref, g2_ref,
             rel_W_ref, rel_b_ref, dic_W_ref, dic_b_ref,
             ln_a_ref, ln_b_ref, lin_W_ref, lin_b_ref,
             fe_W_ref, fe_b_ref,
             Wq_ref, bq_ref, Wk_ref, bk_ref, Wv_ref, bv_ref, Wo_ref, bo_ref,
             anew_ref, pe_ref):
    f = jnp.float32
    rows = rows_ref[0]
    vals = vals_ref[0]
    feats = feats_ref[0]
    arel = arel_ref[0]

    rel_W = rel_W_ref[...]
    dic_W = dic_W_ref[...]
    emb0 = g0_ref[...] @ rel_W + rel_b_ref[...]
    emb1 = g1_ref[...] @ rel_W + rel_b_ref[...]
    emb2 = g2_ref[...] @ rel_W + rel_b_ref[...]
    rg = (emb0 @ dic_W[0:D] + emb1 @ dic_W[D:2 * D]
          + emb2 @ dic_W[2 * D:3 * D] + dic_b_ref[...])

    vmax = jnp.max(vals, axis=0, keepdims=True)
    e = jnp.exp(vals - vmax)
    w = e / jnp.sum(e, axis=0, keepdims=True)

    pe = (rows @ rg) * w

    q = feats @ Wq_ref[...] + bq_ref[...]
    k = pe @ Wk_ref[...] + bk_ref[...]
    v = pe @ Wv_ref[...] + bv_ref[...]

    scale = f(1.0) / jnp.sqrt(f(DH))
    heads = []
    for h in range(H):
        sl = slice(h * DH, (h + 1) * DH)
        s = _nt_dot(q[:, sl], k[:, sl]) * scale
        smax = jnp.max(s, axis=1, keepdims=True)
        es = jnp.exp(s - smax)
        att = es / jnp.sum(es, axis=1, keepdims=True)
        heads.append(att @ v[:, sl])
    mo = jnp.concatenate(heads, axis=1) @ Wo_ref[...] + bo_ref[...]

    mu = jnp.mean(mo, axis=1, keepdims=True)
    xc = mo - mu
    var = jnp.sum(xc * xc, axis=1, keepdims=True) / f(D - 1)
    y = ln_a_ref[...] * xc / (jnp.sqrt(var) + f(1e-6)) + ln_b_ref[...]

    pr = y @ lin_W_ref[...] + lin_b_ref[...]
    fe_W = fe_W_ref[...]
    npre = pr @ fe_W[0:D] + feats @ fe_W[D:2 * D] + fe_b_ref[...]
    anew_ref[0] = _nt_dot(npre, npre) + arel
    pe_ref[0] = pr


def _const2(shape):
    return pl.BlockSpec(shape, lambda b: (0, 0))


@jax.jit
def _tc_call(rows, vals, att_feats, A_rel, g0, g1, g2,
             rel_W, rel_b, dic_W, dic_b, ln_a, ln_b, lin_W, lin_b,
             fe_W, fe_b, Wq, bq, Wk, bk, Wv, bv, Wo, bo):
    specs = [
        pl.BlockSpec((1, TOPN, RPAD), lambda b: (b, 0, 0)),
        pl.BlockSpec((1, TOPN, 1), lambda b: (b, 0, 0)),
        pl.BlockSpec((1, N, D), lambda b: (b, 0, 0)),
        pl.BlockSpec((1, N, N), lambda b: (b, 0, 0)),
        _const2((RPAD, GLOVE)), _const2((RPAD, GLOVE)), _const2((RPAD, GLOVE)),
        _const2((GLOVE, D)), _const2((1, D)),
        _const2((3 * D, D)), _const2((1, D)),
        _const2((1, D)), _const2((1, D)),
        _const2((D, D)), _const2((1, D)),
        _const2((2 * D, D)), _const2((1, D)),
        _const2((D, D)), _const2((1, D)),
        _const2((D, D)), _const2((1, D)),
        _const2((D, D)), _const2((1, D)),
        _const2((D, D)), _const2((1, D)),
    ]
    out_specs = [
        pl.BlockSpec((1, N, N), lambda b: (b, 0, 0)),
        pl.BlockSpec((1, N, D), lambda b: (b, 0, 0)),
    ]
    out_shapes = [
        jax.ShapeDtypeStruct((BS, N, N), jnp.float32),
        jax.ShapeDtypeStruct((BS, N, D), jnp.float32),
    ]
    return pl.pallas_call(
        _tc_body,
        grid=(BS,),
        in_specs=specs,
        out_specs=out_specs,
        out_shape=out_shapes,
    )(rows, vals, att_feats, A_rel, g0, g1, g2,
      rel_W, rel_b, dic_W, dic_b, ln_a, ln_b, lin_W, lin_b,
      fe_W, fe_b, Wq, bq, Wk, bk, Wv, bv, Wo, bo)


def kernel(A_rel, rel_scores, att_feats, att_mask, vocab_glove, rel_dic,
           rel_W, rel_b, dic_W, dic_b, ln_a, ln_b, lin_W, lin_b,
           fe_W, fe_b, Wq, bq, Wk, bk, Wv, bv, Wo, bo):
    del att_mask
    flat = A_rel.reshape(BS, NN)
    vals, idx = lax.top_k(flat, TOPN)
    bi = jnp.arange(BS, dtype=jnp.int32)[:, None]
    rows = rel_scores[bi, idx // N, idx % N, :]
    rows = jnp.pad(rows, ((0, 0), (0, 0), (0, RPAD - R)))

    g = jnp.take(vocab_glove, rel_dic, axis=0)
    g = jnp.pad(g, ((0, RPAD - R), (0, 0), (0, 0)))
    g0, g1, g2 = g[:, 0], g[:, 1], g[:, 2]

    A_new, pe_rel_reps = _tc_call(
        rows, vals[..., None], att_feats, A_rel, g0, g1, g2,
        rel_W, rel_b.reshape(1, D), dic_W, dic_b.reshape(1, D),
        ln_a.reshape(1, D), ln_b.reshape(1, D), lin_W, lin_b.reshape(1, D),
        fe_W, fe_b.reshape(1, D), Wq, bq.reshape(1, D), Wk, bk.reshape(1, D),
        Wv, bv.reshape(1, D), Wo, bo.reshape(1, D))
    return (A_new, pe_rel_reps)
